# Initial kernel scaffold; baseline (speedup 1.0000x reference)
import jax
import jax.numpy as jnp
from jax.experimental import pallas as pl


def kernel(x, edge_index, W0, b0, W1, b1, Wf, bf):
    raise NotImplementedError("write your pallas kernel here")


# speedup vs baseline: 56.1250x; 56.1250x over previous
import functools

import jax
import jax.numpy as jnp
from jax.experimental import pallas as pl
from jax.experimental.pallas import tpu as pltpu


def _ceil_to(x, m):
  return ((x + m - 1) // m) * m


def _quad_body(nk, v_ref, a_ref, quad_ref, rs_ref, acc_ref):
  j = pl.program_id(1)
  k = pl.program_id(2)
  bk = a_ref.shape[0]
  bj = a_ref.shape[1]

  @pl.when(k == 0)
  def _():
    acc_ref[...] = jnp.zeros_like(acc_ref)

  vk = v_ref[:, pl.ds(k * bk, bk)]
  acc_ref[...] += jax.lax.dot_general(
      vk, a_ref[...], (((1,), (0,)), ((), ())),
      preferred_element_type=jnp.float32)

  @pl.when(jnp.logical_and(j == 0, k == 0))
  def _():
    rs_ref[...] = jnp.zeros_like(rs_ref)

  @pl.when(j == 0)
  def _():
    rs_ref[0, 0, :] += jnp.sum(vk.astype(jnp.float32), axis=1)

  @pl.when(k == nk - 1)
  def _():
    vj = v_ref[:, pl.ds(j * bj, bj)].astype(jnp.float32)
    part = jnp.sum(acc_ref[...] * vj, axis=1)

    @pl.when(j == 0)
    def _():
      quad_ref[0, 0, :] = part

    @pl.when(j > 0)
    def _():
      quad_ref[0, 0, :] += part


def _quad_call(V, A, nchunk, c, bk, bj):
  np_ = A.shape[0]
  nj = np_ // bj
  nk = np_ // bk
  return pl.pallas_call(
      functools.partial(_quad_body, nk),
      grid=(nchunk, nj, nk),
      in_specs=[
          pl.BlockSpec((c, np_), lambda ci, j, k: (ci, 0)),
          pl.BlockSpec((bk, bj), lambda ci, j, k: (k, j)),
      ],
      out_specs=[
          pl.BlockSpec((1, 1, c), lambda ci, j, k: (ci, 0, 0)),
          pl.BlockSpec((1, 1, c), lambda ci, j, k: (ci, 0, 0)),
      ],
      out_shape=[
          jax.ShapeDtypeStruct((nchunk, 1, c), jnp.float32),
          jax.ShapeDtypeStruct((nchunk, 1, c), jnp.float32),
      ],
      scratch_shapes=[pltpu.VMEM((c, bj), jnp.float32)],
      compiler_params=pltpu.CompilerParams(
          dimension_semantics=("parallel", "arbitrary", "arbitrary")),
  )(V, A)


def _segmax_body(ce, pk_ref, nr_ref, y_ref, m_ref):

  @pl.when(pl.program_id(0) == 0)
  def _():
    m_ref[...] = jnp.full_like(m_ref, -jnp.inf)

  def step(e, carry):
    pkv = pk_ref[0, 0, e]
    s = pkv >> 14
    d = pkv & 16383
    nr = nr_ref[0, 0, e]
    row = y_ref[pl.ds(s, 1), :]
    m_ref[pl.ds(d, 1), :] = jnp.maximum(m_ref[pl.ds(d, 1), :], nr * row)
    return carry

  jax.lax.fori_loop(0, ce, step, 0)


def _segmax_call(pk, nr, y, n_out, nchunk, ce):
  d = y.shape[1]
  return pl.pallas_call(
      functools.partial(_segmax_body, ce),
      grid=(nchunk,),
      in_specs=[
          pl.BlockSpec((1, 1, ce), lambda c: (c, 0, 0),
                       memory_space=pltpu.SMEM),
          pl.BlockSpec((1, 1, ce), lambda c: (c, 0, 0),
                       memory_space=pltpu.SMEM),
          pl.BlockSpec(y.shape, lambda c: (0, 0)),
      ],
      out_specs=pl.BlockSpec((n_out, d), lambda c: (0, 0)),
      out_shape=jax.ShapeDtypeStruct((n_out, d), jnp.float32),
      compiler_params=pltpu.CompilerParams(
          dimension_semantics=("arbitrary",)),
  )(pk, nr, y)


def _lin_body(h_ref, w_ref, o_ref):
  o_ref[...] = jax.lax.dot_general(
      h_ref[...], w_ref[...], (((1,), (1,)), ((), ())),
      preferred_element_type=jnp.float32)


def _lin_call(h, w, rb):
  n, _ = h.shape
  dout = w.shape[0]
  return pl.pallas_call(
      _lin_body,
      grid=(n // rb,),
      in_specs=[
          pl.BlockSpec((rb, h.shape[1]), lambda i: (i, 0)),
          pl.BlockSpec(w.shape, lambda i: (0, 0)),
      ],
      out_specs=pl.BlockSpec((rb, dout), lambda i: (i, 0)),
      out_shape=jax.ShapeDtypeStruct((n, dout), jnp.float32),
  )(h, w)


def _lin_relu_body(h_ref, b_ref, w_ref, o_ref):
  a = jnp.maximum(h_ref[...] + b_ref[...], 0.0)
  o_ref[...] = jax.lax.dot_general(
      a, w_ref[...], (((1,), (1,)), ((), ())),
      preferred_element_type=jnp.float32)


def _lin_relu_call(h, b, w, rb):
  n, _ = h.shape
  dout = w.shape[0]
  return pl.pallas_call(
      _lin_relu_body,
      grid=(n // rb,),
      in_specs=[
          pl.BlockSpec((rb, h.shape[1]), lambda i: (i, 0)),
          pl.BlockSpec((1, h.shape[1]), lambda i: (0, 0)),
          pl.BlockSpec(w.shape, lambda i: (0, 0)),
      ],
      out_specs=pl.BlockSpec((rb, dout), lambda i: (i, 0)),
      out_shape=jax.ShapeDtypeStruct((n, dout), jnp.float32),
  )(h, b, w)


def _final_body(h_ref, b_ref, wf_ref, bf_ref, o_ref):
  a = jnp.maximum(h_ref[...] + b_ref[...], 0.0)
  logits = jax.lax.dot_general(
      a, wf_ref[...], (((1,), (1,)), ((), ())),
      preferred_element_type=jnp.float32) + bf_ref[...]
  mx = jnp.max(logits, axis=-1, keepdims=True)
  lse = mx + jnp.log(jnp.sum(jnp.exp(logits - mx), axis=-1, keepdims=True))
  o_ref[...] = logits - lse


def _final_call(h, b, wf, bf, rb):
  n, _ = h.shape
  ncls = wf.shape[0]
  return pl.pallas_call(
      _final_body,
      grid=(n // rb,),
      in_specs=[
          pl.BlockSpec((rb, h.shape[1]), lambda i: (i, 0)),
          pl.BlockSpec((1, h.shape[1]), lambda i: (0, 0)),
          pl.BlockSpec(wf.shape, lambda i: (0, 0)),
          pl.BlockSpec((1, ncls), lambda i: (0, 0)),
      ],
      out_specs=pl.BlockSpec((rb, ncls), lambda i: (i, 0)),
      out_shape=jax.ShapeDtypeStruct((n, ncls), jnp.float32),
  )(h, b, wf, bf)


def kernel(x, edge_index, W0, b0, W1, b1, Wf, bf):
  n, d_in = x.shape
  e = edge_index.shape[1]
  ef = e + n
  np_ = _ceil_to(n, 512)

  diag = jnp.arange(n, dtype=jnp.int32)
  ei0 = edge_index[0].astype(jnp.int32)
  ei1 = edge_index[1].astype(jnp.int32)

  A = jnp.zeros((np_, np_), dtype=jnp.bfloat16)
  A = A.at[ei0, ei1].set(jnp.bfloat16(1.0))
  A = A.at[ei1, ei0].set(jnp.bfloat16(1.0))
  A = A.at[diag, diag].set(jnp.bfloat16(1.0))

  c = 1024
  epad = _ceil_to(ef, c)
  nchunk = epad // c
  zpad = jnp.zeros((epad - ef,), dtype=jnp.int32)
  srcf = jnp.concatenate([ei0, diag, zpad])
  dstf = jnp.concatenate([ei1, diag, zpad])

  V = jnp.take(A, srcf, axis=0) * jnp.take(A, dstf, axis=0)

  bk = bj = min(512, np_)
  quad, rs = _quad_call(V, A, nchunk, c, bk, bj)
  quad = quad.reshape(-1)[:ef]
  rs = rs.reshape(-1)[:ef]

  inter = 0.5 * (quad + rs)
  ecnt = inter[e:]
  src = srcf[:ef]
  dst = dstf[:ef]
  uni = ecnt[src] + ecnt[dst] - inter
  norm = jnp.where(uni > 0, inter / uni, jnp.float32(0.0))

  ce = 1000 if ef % 1000 == 0 else ef
  echunk = _ceil_to(ef, ce)
  nechunk = echunk // ce
  n_out = n + 8
  pad_e = echunk - ef
  pk = jnp.concatenate([
      (src << 14) | dst,
      jnp.full((pad_e,), n, dtype=jnp.int32),
  ]).reshape(nechunk, 1, ce)
  nrm = jnp.concatenate([norm, jnp.zeros((pad_e,), jnp.float32)])
  nrm = nrm.reshape(nechunk, 1, ce)

  rb = 1000 if n % 1000 == 0 else n

  h = _lin_call(x, W0, rb)
  m = _segmax_call(pk, nrm, h, n_out, nechunk, ce)[:n]
  h = _lin_relu_call(m, b0.reshape(1, -1), W1, rb)
  m = _segmax_call(pk, nrm, h, n_out, nechunk, ce)[:n]
  out = _final_call(m, b1.reshape(1, -1), Wf, bf.reshape(1, -1), rb)
  return out
